# baseline (speedup 1.0000x reference)
import functools

import jax
import jax.numpy as jnp
from jax import lax
from jax.experimental import pallas as pl
from jax.experimental.pallas import tpu as pltpu
from jax.experimental.pallas import tpu_sc as plsc

N = 65536
H = 1024
NUM_ELEMENTS = 100
B = 1024
T = 1024
EPAD = 128
CH = 4
TC_ = T // CH

NSUB = 16
CHUNK = N // NSUB
ROWS = CHUNK // 128


def _mlp_body(z_ref, pos_ref, emb_ref, b1_ref, Wp_ref, W1_ref, W2_ref,
              b2_ref, y_ref, E1_ref):
    @pl.when(pl.program_id(0) == 0)
    def _():
        E1_ref[...] = (jnp.dot(emb_ref[...].astype(jnp.bfloat16), W1_ref[...],
                               preferred_element_type=jnp.float32)
                       + b1_ref[...]).astype(jnp.bfloat16)
    W2r = W2_ref[...]
    pas = []
    for c in range(CH):
        rs = pl.ds(c * TC_, TC_)
        pas.append(jnp.dot(pos_ref[rs, :], Wp_ref[...],
                           preferred_element_type=jnp.float32))
    xps = [jnp.tanh(pa).astype(jnp.bfloat16) for pa in pas]

    hs = [None] * CH
    for c in range(CH + 1):
        if c < CH:
            rs = pl.ds(c * TC_, TC_)
            z = z_ref[rs, :]
            onehot_z = (z == lax.broadcasted_iota(jnp.int32, (TC_, EPAD), 1)
                        ).astype(jnp.bfloat16)
            hs[c] = (jnp.dot(onehot_z, E1_ref[...],
                             preferred_element_type=jnp.float32)
                     + jnp.dot(xps[c], W1_ref[...],
                               preferred_element_type=jnp.float32))
        if c > 0:
            h = hs[c - 1]
            h = h * (0.5 * jnp.tanh(0.5 * h) + 0.5)
            y = jnp.sum(h * W2r, axis=1, keepdims=True)
            y_ref[pl.ds((c - 1) * TC_, TC_), :] = y + b2_ref[0, 0]


def _atom_mlp(z, pos, emb, Wp, W1, b1, W2, b2, interpret=False):
    z2 = z.reshape(N, 1).astype(jnp.int32)
    embp = jnp.zeros((EPAD, H), jnp.float32).at[:NUM_ELEMENTS].set(emb)
    W1b = W1.astype(jnp.bfloat16)
    Wpb = Wp.astype(jnp.bfloat16)
    posb = pos.astype(jnp.bfloat16)
    b1r = b1.reshape(1, H // 2)
    W2r = W2.reshape(1, H // 2)
    b2r = jnp.asarray(b2, jnp.float32).reshape(1, 1)

    return pl.pallas_call(
        _mlp_body,
        grid=(N // T,),
        in_specs=[
            pl.BlockSpec((T, 1), lambda i: (i, 0)),
            pl.BlockSpec((T, 3), lambda i: (i, 0)),
            pl.BlockSpec((EPAD, H), lambda i: (0, 0)),
            pl.BlockSpec((1, H // 2), lambda i: (0, 0)),
            pl.BlockSpec((3, H), lambda i: (0, 0)),
            pl.BlockSpec((H, H // 2), lambda i: (0, 0)),
            pl.BlockSpec((1, H // 2), lambda i: (0, 0)),
            pl.BlockSpec(memory_space=pltpu.SMEM),
        ],
        out_specs=pl.BlockSpec((T, 1), lambda i: (i, 0)),
        out_shape=jax.ShapeDtypeStruct((N, 1), jnp.float32),
        scratch_shapes=[pltpu.VMEM((EPAD, H // 2), jnp.bfloat16)],
        interpret=interpret,
    )(z2, posb, embp, b1r, Wpb, W1b, W2r, b2r)


def _pool_body(y_hbm, idx_hbm, std_hbm, mean_hbm, out_hbm,
               y_v, idx_v, zero_v, acc_shared, tmp_v, scl_v):
    cid = lax.axis_index("c")
    sid = lax.axis_index("s")

    @pl.when(cid == 0)
    def _():
        pltpu.sync_copy(y_hbm.at[sid], y_v)
        pltpu.sync_copy(idx_hbm.at[sid], idx_v)

        @pl.when(sid == 0)
        def _():
            for j in range(B // 16):
                zero_v[pl.ds(j * 16, 16)] = jnp.zeros((16,), jnp.float32)
            pltpu.sync_copy(zero_v, acc_shared)

        plsc.subcore_barrier()
        for j in range(ROWS):
            pltpu.sync_copy(y_v.at[j], acc_shared.at[idx_v.at[j]], add=True)
        plsc.subcore_barrier()

        pltpu.sync_copy(std_hbm, scl_v.at[0])
        pltpu.sync_copy(mean_hbm, scl_v.at[1])
        pltpu.sync_copy(acc_shared.at[pl.ds(sid * (B // NSUB), B // NSUB)],
                        tmp_v)
        stdv = scl_v[0, :]
        meanv = scl_v[1, :]
        for j in range(B // NSUB // 16):
            sl = pl.ds(j * 16, 16)
            tmp_v[sl] = tmp_v[sl] * stdv + meanv
        pltpu.sync_copy(tmp_v, out_hbm.at[pl.ds(sid * (B // NSUB), B // NSUB)])


def _make_pool():
    return functools.partial(
        pl.kernel,
        mesh=plsc.VectorSubcoreMesh(core_axis_name="c", subcore_axis_name="s"),
        out_type=jax.ShapeDtypeStruct((B,), jnp.float32),
        scratch_types=[
            pltpu.VMEM((ROWS, 128), jnp.float32),
            pltpu.VMEM((ROWS, 128), jnp.int32),
            pltpu.VMEM((B,), jnp.float32),
            pltpu.VMEM_SHARED((B,), jnp.float32),
            pltpu.VMEM((B // NSUB,), jnp.float32),
            pltpu.VMEM((2, 16), jnp.float32),
        ],
    )(_pool_body)


@jax.jit
def kernel(z, pos, batch, emb, Wp, W1, b1, W2, b2, mean, std):
    y = _atom_mlp(z, pos, emb, Wp, W1, b1, W2, b2)
    y3 = y.reshape(NSUB, ROWS, 128)
    idx3 = batch.astype(jnp.int32).reshape(NSUB, ROWS, 128)
    stdv = jnp.full((16,), std, jnp.float32)
    meanv = jnp.full((16,), mean, jnp.float32)
    out = _make_pool()(y3, idx3, stdv, meanv)
    return out.reshape(B, 1)


# speedup vs baseline: 1.0746x; 1.0746x over previous
import functools

import jax
import jax.numpy as jnp
from jax import lax
from jax.experimental import pallas as pl
from jax.experimental.pallas import tpu as pltpu
from jax.experimental.pallas import tpu_sc as plsc

N = 65536
H = 1024
NUM_ELEMENTS = 100
B = 1024
T = 2048
EPAD = 128
CH = 4
TC_ = T // CH

NSUB = 16
CHUNK = N // NSUB
ROWS = CHUNK // 128


def _fold_body(emb_ref, W1_ref, b1_ref, E1_ref):
    E1_ref[...] = (jnp.dot(emb_ref[...].astype(jnp.bfloat16), W1_ref[...],
                           preferred_element_type=jnp.float32)
                   + b1_ref[...]).astype(jnp.bfloat16)


def _mlp_body(z_ref, pos_ref, E1_ref, Wp_ref, W1_ref, W2_ref, b2_ref,
              y_ref):
    W2r = W2_ref[...]
    pas = []
    for c in range(CH):
        rs = pl.ds(c * TC_, TC_)
        pas.append(jnp.dot(pos_ref[rs, :], Wp_ref[...],
                           preferred_element_type=jnp.float32))
    xps = [jnp.tanh(pa).astype(jnp.bfloat16) for pa in pas]

    hs = [None] * CH
    for c in range(CH + 1):
        if c < CH:
            rs = pl.ds(c * TC_, TC_)
            z = z_ref[rs, :]
            onehot_z = (z == lax.broadcasted_iota(jnp.int32, (TC_, EPAD), 1)
                        ).astype(jnp.bfloat16)
            hs[c] = (jnp.dot(onehot_z, E1_ref[...],
                             preferred_element_type=jnp.float32)
                     + jnp.dot(xps[c], W1_ref[...],
                               preferred_element_type=jnp.float32))
        if c > 0:
            h = hs[c - 1]
            h = h * (0.5 * jnp.tanh(0.5 * h) + 0.5)
            y = jnp.sum(h * W2r, axis=1, keepdims=True)
            y_ref[pl.ds((c - 1) * TC_, TC_), :] = y + b2_ref[0, 0]


def _atom_mlp(z, pos, emb, Wp, W1, b1, W2, b2, interpret=False):
    z2 = z.reshape(N, 1).astype(jnp.int32)
    embp = jnp.zeros((EPAD, H), jnp.float32).at[:NUM_ELEMENTS].set(emb)
    W1b = W1.astype(jnp.bfloat16)
    Wpb = Wp.astype(jnp.bfloat16)
    posb = pos.astype(jnp.bfloat16)
    b1r = b1.reshape(1, H // 2)
    W2r = W2.reshape(1, H // 2)
    b2r = jnp.asarray(b2, jnp.float32).reshape(1, 1)

    E1 = pl.pallas_call(
        _fold_body,
        in_specs=[
            pl.BlockSpec((EPAD, H), lambda: (0, 0)),
            pl.BlockSpec((H, H // 2), lambda: (0, 0)),
            pl.BlockSpec((1, H // 2), lambda: (0, 0)),
        ],
        out_specs=pl.BlockSpec((EPAD, H // 2), lambda: (0, 0)),
        out_shape=jax.ShapeDtypeStruct((EPAD, H // 2), jnp.bfloat16),
        interpret=interpret,
    )(embp, W1b, b1r)

    return pl.pallas_call(
        _mlp_body,
        grid=(N // T,),
        in_specs=[
            pl.BlockSpec((T, 1), lambda i: (i, 0)),
            pl.BlockSpec((T, 3), lambda i: (i, 0)),
            pl.BlockSpec((EPAD, H // 2), lambda i: (0, 0)),
            pl.BlockSpec((3, H), lambda i: (0, 0)),
            pl.BlockSpec((H, H // 2), lambda i: (0, 0)),
            pl.BlockSpec((1, H // 2), lambda i: (0, 0)),
            pl.BlockSpec(memory_space=pltpu.SMEM),
        ],
        out_specs=pl.BlockSpec((T, 1), lambda i: (i, 0)),
        out_shape=jax.ShapeDtypeStruct((N, 1), jnp.float32),
        interpret=interpret,
    )(z2, posb, E1, Wpb, W1b, W2r, b2r)


def _pool_body(y_hbm, idx_hbm, std_hbm, mean_hbm, out_hbm,
               y_v, idx_v, zero_v, acc_shared, tmp_v, scl_v, sem):
    cid = lax.axis_index("c")
    sid = lax.axis_index("s")

    @pl.when(cid == 0)
    def _():
        pltpu.sync_copy(y_hbm.at[sid], y_v)
        pltpu.sync_copy(idx_hbm.at[sid], idx_v)

        @pl.when(sid == 0)
        def _():
            for j in range(B // 16):
                zero_v[pl.ds(j * 16, 16)] = jnp.zeros((16,), jnp.float32)
            pltpu.sync_copy(zero_v, acc_shared)

        plsc.subcore_barrier()
        handles = [
            pltpu.async_copy(y_v.at[j], acc_shared.at[idx_v.at[j]], sem,
                             add=True)
            for j in range(ROWS)
        ]
        for hd in handles:
            hd.wait()
        plsc.subcore_barrier()

        pltpu.sync_copy(std_hbm, scl_v.at[0])
        pltpu.sync_copy(mean_hbm, scl_v.at[1])
        pltpu.sync_copy(acc_shared.at[pl.ds(sid * (B // NSUB), B // NSUB)],
                        tmp_v)
        stdv = scl_v[0, :]
        meanv = scl_v[1, :]
        for j in range(B // NSUB // 16):
            sl = pl.ds(j * 16, 16)
            tmp_v[sl] = tmp_v[sl] * stdv + meanv
        pltpu.sync_copy(tmp_v, out_hbm.at[pl.ds(sid * (B // NSUB), B // NSUB)])


def _make_pool():
    return functools.partial(
        pl.kernel,
        mesh=plsc.VectorSubcoreMesh(core_axis_name="c", subcore_axis_name="s"),
        out_type=jax.ShapeDtypeStruct((B,), jnp.float32),
        scratch_types=[
            pltpu.VMEM((ROWS, 128), jnp.float32),
            pltpu.VMEM((ROWS, 128), jnp.int32),
            pltpu.VMEM((B,), jnp.float32),
            pltpu.VMEM_SHARED((B,), jnp.float32),
            pltpu.VMEM((B // NSUB,), jnp.float32),
            pltpu.VMEM((2, 16), jnp.float32),
            pltpu.SemaphoreType.DMA,
        ],
    )(_pool_body)


@jax.jit
def kernel(z, pos, batch, emb, Wp, W1, b1, W2, b2, mean, std):
    y = _atom_mlp(z, pos, emb, Wp, W1, b1, W2, b2)
    y3 = y.reshape(NSUB, ROWS, 128)
    idx3 = batch.astype(jnp.int32).reshape(NSUB, ROWS, 128)
    stdv = jnp.full((16,), std, jnp.float32)
    meanv = jnp.full((16,), mean, jnp.float32)
    out = _make_pool()(y3, idx3, stdv, meanv)
    return out.reshape(B, 1)
